# baseline (speedup 1.0000x reference)
import jax
import jax.numpy as jnp
import numpy as np
from jax.experimental import pallas as pl
from jax.experimental.pallas import tpu as pltpu

_BM = 4096


def _proj_kernel(idx_ref, wt_ref, x_ref, out_ref, p_ref):
    j = pl.program_id(1)

    @pl.when((pl.program_id(0) == 0) & (j == 0))
    def _():
        K = wt_ref.shape[1]
        C = 256
        kio = jax.lax.broadcasted_iota(jnp.int32, (K, C), 0)
        s = jnp.zeros((K, C), jnp.float32)
        for jj in range(idx_ref.shape[0]):
            s = s + (kio == idx_ref[jj : jj + 1, :]).astype(jnp.float32)
        p = jnp.dot(wt_ref[...], s, preferred_element_type=jnp.float32)
        inv_scale = np.float32(1.0 / np.sqrt(float(wt_ref.shape[0])))
        p_ref[...] = p * inv_scale

    out_ref[...] = jnp.dot(x_ref[...], p_ref[...],
                           preferred_element_type=jnp.float32)


def kernel(inputs, weight, feature_table):
    B, E = inputs.shape
    T = weight.shape[0]
    V, F = feature_table.shape
    C = V + 1

    wt = jnp.concatenate([weight, jnp.zeros((1, E), weight.dtype)], axis=0).T

    ftT = feature_table.T.astype(jnp.int32)
    pad_row = jnp.full((1, V), T, jnp.int32)
    ftT8 = jnp.concatenate([ftT, pad_row], axis=0)
    col0 = jnp.full((F + 1, 1), T, jnp.int32).at[0, 0].set(0)
    lanes_pad = jnp.full((F + 1, 256 - C), T, jnp.int32)
    idx = jnp.concatenate([col0, ftT8, lanes_pad], axis=1)

    grid = (B // _BM, 1)
    out = pl.pallas_call(
        _proj_kernel,
        grid=grid,
        in_specs=[
            pl.BlockSpec((F + 1, 256), lambda i, j: (0, 0)),
            pl.BlockSpec((E, T + 1), lambda i, j: (0, 0)),
            pl.BlockSpec((_BM, E), lambda i, j: (i, 0)),
        ],
        out_specs=pl.BlockSpec((_BM, 256), lambda i, j: (i, j)),
        out_shape=jax.ShapeDtypeStruct((B, 256), jnp.float32),
        scratch_shapes=[pltpu.VMEM((E, 256), jnp.float32)],
    )(idx, wt, inputs)
    return jax.lax.slice(out, (0, 0), (B, C))


# speedup vs baseline: 1.5756x; 1.5756x over previous
import jax
import jax.numpy as jnp
import numpy as np
from jax.experimental import pallas as pl
from jax.experimental.pallas import tpu as pltpu

_BM = 4096


def _tc_body(idx_ref, wt_ref, x_ref, o_ref, p_ref, buf, buf_e, sem_b, sem_e):
    i = pl.program_id(0)
    n = pl.num_programs(0)
    nb = buf.shape[0]
    slot = lax_rem = i % nb

    @pl.when(i == 0)
    def _():
        K = wt_ref.shape[1]
        C2 = p_ref.shape[1]
        kio = jax.lax.broadcasted_iota(jnp.int32, (K, C2), 0)
        s = jnp.zeros((K, C2), jnp.float32)
        for j in range(idx_ref.shape[0]):
            s = s + (kio == idx_ref[j : j + 1, :]).astype(jnp.float32)
        p = jnp.dot(wt_ref[...], s, preferred_element_type=jnp.float32)
        inv_scale = np.float32(1.0 / np.sqrt(float(wt_ref.shape[0])))
        p_ref[...] = p * inv_scale

    def copies(step, sl):
        row = step * _BM
        bulk = pltpu.make_async_copy(
            buf.at[sl],
            o_ref.at[pl.ds(row, _BM), pl.ds(0, 128)],
            sem_b.at[sl],
        )
        edge = pltpu.make_async_copy(
            buf_e.at[sl],
            o_ref.at[pl.ds(row, _BM), pl.ds(128, 1)],
            sem_e.at[sl],
        )
        return bulk, edge

    @pl.when(i >= nb)
    def _():
        bulk, edge = copies(i - nb, slot)
        bulk.wait()
        edge.wait()

    res = jnp.dot(x_ref[...], p_ref[...], preferred_element_type=jnp.float32)
    buf[slot] = res[:, :128]
    buf_e[slot] = res[:, 128:129]

    bulk, edge = copies(i, slot)
    edge.start()
    bulk.start()

    @pl.when(i == n - 1)
    def _():
        for d in range(nb):
            step = i - d
            @pl.when(step >= 0)
            def _():
                bulk, edge = copies(step, (i - d) % nb)
                bulk.wait()
                edge.wait()


def kernel(inputs, weight, feature_table):
    B, E = inputs.shape
    T = weight.shape[0]
    V, F = feature_table.shape
    C = V + 1

    wt = jnp.concatenate([weight, jnp.zeros((1, E), weight.dtype)], axis=0).T

    ftT = feature_table.T.astype(jnp.int32)
    pad_row = jnp.full((1, V), T, jnp.int32)
    ftT8 = jnp.concatenate([ftT, pad_row], axis=0)
    col0 = jnp.full((F + 1, 1), T, jnp.int32).at[0, 0].set(0)
    lanes_pad = jnp.full((F + 1, 2 * E - C), T, jnp.int32)
    idx = jnp.concatenate([col0, ftT8, lanes_pad], axis=1)

    grid = (B // _BM,)
    return pl.pallas_call(
        _tc_body,
        grid=grid,
        in_specs=[
            pl.BlockSpec((F + 1, 2 * E), lambda i: (0, 0)),
            pl.BlockSpec((E, T + 1), lambda i: (0, 0)),
            pl.BlockSpec((_BM, E), lambda i: (i, 0)),
        ],
        out_specs=pl.BlockSpec(memory_space=pltpu.HBM),
        out_shape=jax.ShapeDtypeStruct((B, C), jnp.float32),
        scratch_shapes=[
            pltpu.VMEM((E, 2 * E), jnp.float32),
            pltpu.VMEM((2, _BM, E), jnp.float32),
            pltpu.VMEM((2, _BM, 1), jnp.float32),
            pltpu.SemaphoreType.DMA((2,)),
            pltpu.SemaphoreType.DMA((2,)),
        ],
    )(idx, wt, inputs)
